# baseline (speedup 1.0000x reference)
import jax
import jax.numpy as jnp
from jax import lax
from jax.experimental import pallas as pl
from jax.experimental.pallas import tpu as pltpu
from jax.experimental.pallas import tpu_sc as plsc

N = 10000
E = 320000
D = 128

NC = 2
NS = 16
NW = NC * NS
EPW = E // NW
CHUNK = 80
NCHUNK = EPW // CHUNK
MS = 6
P = 10240
RPT = P // NS
POFF = 12000

_CB = 2000


def _matmul_body(x_ref, w_ref, o_ref):
    o_ref[...] = jnp.dot(x_ref[...], w_ref[...],
                         preferred_element_type=jnp.float32)


def _matmul(x, W):
    return pl.pallas_call(
        _matmul_body,
        grid=(N // _CB,),
        in_specs=[
            pl.BlockSpec((_CB, D), lambda i: (i, 0)),
            pl.BlockSpec((D, D), lambda i: (0, 0)),
        ],
        out_specs=pl.BlockSpec((_CB, D), lambda i: (i, 0)),
        out_shape=jax.ShapeDtypeStruct((N, D), jnp.float32),
    )(x, W)


def _add_body(a_ref, b_ref, o_ref):
    o_ref[...] = a_ref[...] + b_ref[...]


def _combine(p):
    return pl.pallas_call(
        _add_body,
        grid=(N // _CB,),
        in_specs=[
            pl.BlockSpec((_CB, D), lambda i: (i, 0)),
            pl.BlockSpec((_CB, D), lambda i: (i + POFF // _CB, 0)),
        ],
        out_specs=pl.BlockSpec((_CB, D), lambda i: (i, 0)),
        out_shape=jax.ShapeDtypeStruct((N, D), jnp.float32),
    )(p, p)


def _spmm_body(h_hbm, src_hbm, dst_hbm, w_hbm, p_hbm,
               rows, src_m, dst_m, w_m, acc_sh,
               gsems, ssems, msems):
    cid = lax.axis_index("c")
    sid = lax.axis_index("s")
    wid = cid * NS + sid

    def _zrow(r, _):
        for k in range(D // 16):
            rows[0, r, pl.ds(k * 16, 16)] = jnp.zeros((16,), jnp.float32)
        return ()
    lax.fori_loop(0, CHUNK, _zrow, ())
    for b in range(RPT // CHUNK):
        pltpu.sync_copy(rows.at[0],
                        acc_sh.at[pl.ds(sid * RPT + b * CHUNK, CHUNK)])
    plsc.subcore_barrier()

    def m_start(j, slot):
        ed = pl.ds(wid * EPW + j * CHUNK, CHUNK)
        pltpu.async_copy(src_hbm.at[ed], src_m.at[slot], msems.at[slot])
        pltpu.async_copy(dst_hbm.at[ed], dst_m.at[slot], msems.at[slot])
        pltpu.async_copy(w_hbm.at[ed], w_m.at[slot], msems.at[slot])

    def m_wait(j, slot):
        ed = pl.ds(wid * EPW + j * CHUNK, CHUNK)
        pltpu.make_async_copy(src_hbm.at[ed], src_m.at[slot],
                              msems.at[slot]).wait()
        pltpu.make_async_copy(dst_hbm.at[ed], dst_m.at[slot],
                              msems.at[slot]).wait()
        pltpu.make_async_copy(w_hbm.at[ed], w_m.at[slot],
                              msems.at[slot]).wait()

    def g_copy(mslot, buf_slot):
        return pltpu.make_async_copy(
            h_hbm.at[src_m.at[mslot]], rows.at[buf_slot], gsems.at[buf_slot])

    def s_start(buf_slot, mslot):
        pltpu.async_copy(rows.at[buf_slot], acc_sh.at[dst_m.at[mslot]],
                         ssems.at[buf_slot], add=True)

    def s_wait(buf_slot, mslot):
        pltpu.make_async_copy(rows.at[buf_slot], acc_sh.at[dst_m.at[mslot]],
                              ssems.at[buf_slot]).wait()

    def scale(buf_slot, mslot):
        def _group(g, _):
            w16 = w_m[mslot, pl.ds(g * 16, 16)]
            for l in range(16):
                ws = jnp.broadcast_to(w16[l], (16,))
                e = g * 16 + l
                for k in range(D // 16):
                    sl = pl.ds(k * 16, 16)
                    rows[buf_slot, e, sl] = rows[buf_slot, e, sl] * ws
            return ()
        lax.fori_loop(0, CHUNK // 16, _group, ())

    for j0 in range(5):
        m_start(j0, j0)
    for j0 in range(3):
        m_wait(j0, j0)
        g_copy(j0, j0).start()

    def _step(j, _):
        bs = lax.rem(j, 4)
        nbs = lax.rem(j + 3, 4)
        ms = lax.rem(j, MS)
        g_copy(ms, bs).wait()
        scale(bs, ms)
        s_start(bs, ms)

        @pl.when(j > 0)
        def _():
            s_wait(nbs, lax.rem(j - 1, MS))

        @pl.when(j + 5 < NCHUNK)
        def _():
            m_start(j + 5, lax.rem(j + 5, MS))

        @pl.when(j + 3 < NCHUNK)
        def _():
            nms = lax.rem(j + 3, MS)
            m_wait(j + 3, nms)
            g_copy(nms, nbs).start()
        return ()
    lax.fori_loop(0, NCHUNK, _step, ())

    s_wait(lax.rem(NCHUNK - 1, 4), lax.rem(NCHUNK - 1, MS))

    plsc.subcore_barrier()
    pltpu.sync_copy(acc_sh.at[pl.ds(sid * RPT, RPT)],
                    p_hbm.at[pl.ds(cid * POFF + sid * RPT, RPT)])


def _spmm_sc(h, src, dst, w):
    mesh = plsc.VectorSubcoreMesh(core_axis_name="c", subcore_axis_name="s")
    return pl.kernel(
        _spmm_body,
        out_type=jax.ShapeDtypeStruct((2 * POFF, D), jnp.float32),
        mesh=mesh,
        compiler_params=pltpu.CompilerParams(needs_layout_passes=False),
        scratch_types=[
            pltpu.VMEM((4, CHUNK, D), jnp.float32),
            pltpu.VMEM((MS, CHUNK), jnp.int32),
            pltpu.VMEM((MS, CHUNK), jnp.int32),
            pltpu.VMEM((MS, CHUNK), jnp.float32),
            pltpu.VMEM_SHARED((P, D), jnp.float32),
            pltpu.SemaphoreType.DMA((4,)),
            pltpu.SemaphoreType.DMA((4,)),
            pltpu.SemaphoreType.DMA((MS,)),
        ],
    )(h, src, dst, w)


def kernel(x, edge_index, edge_weight, W):
    src = edge_index[0]
    dst = edge_index[1]

    h = _matmul(x, W)
    y = _combine(_spmm_sc(h, src, dst, edge_weight))
    out = _combine(_spmm_sc(y, src, dst, edge_weight))
    return out


# speedup vs baseline: 2.5271x; 2.5271x over previous
import jax
import jax.numpy as jnp
from jax import lax
from jax.experimental import pallas as pl
from jax.experimental.pallas import tpu as pltpu
from jax.experimental.pallas import tpu_sc as plsc

N = 10000
E = 320000
D = 128

NC = 2
NS = 16
NW = NC * NS
EPW = E // NW
CHUNK = 80
NCHUNK = EPW // CHUNK
MS = 6
P = 10240
RPT = P // NS
POFF = 12000

_CB = 2000


def _matmul_body(x_ref, w_ref, o_ref):
    o_ref[...] = jnp.dot(x_ref[...], w_ref[...],
                         preferred_element_type=jnp.float32)


def _matmul(x, W):
    return pl.pallas_call(
        _matmul_body,
        grid=(N // _CB,),
        in_specs=[
            pl.BlockSpec((_CB, D), lambda i: (i, 0)),
            pl.BlockSpec((D, D), lambda i: (0, 0)),
        ],
        out_specs=pl.BlockSpec((_CB, D), lambda i: (i, 0)),
        out_shape=jax.ShapeDtypeStruct((N, D), jnp.float32),
    )(x, W)


def _add_body(a_ref, b_ref, o_ref):
    o_ref[...] = a_ref[...] + b_ref[...]


def _combine(p):
    return pl.pallas_call(
        _add_body,
        grid=(N // _CB,),
        in_specs=[
            pl.BlockSpec((_CB, D), lambda i: (i, 0)),
            pl.BlockSpec((_CB, D), lambda i: (i + POFF // _CB, 0)),
        ],
        out_specs=pl.BlockSpec((_CB, D), lambda i: (i, 0)),
        out_shape=jax.ShapeDtypeStruct((N, D), jnp.float32),
    )(p, p)


def _spmm_body(h_hbm, src_hbm, dst_hbm, w_hbm, p_hbm,
               rows_a, rows_b, scaled_a, scaled_b,
               src_m, dst_m, w_m, acc_sh,
               gsem_a, gsem_b, ssem_a, ssem_b, msems):
    cid = lax.axis_index("c")
    sid = lax.axis_index("s")
    wid = cid * NS + sid

    def m_start(j, slot):
        ed = pl.ds(wid * EPW + j * CHUNK, CHUNK)
        pltpu.async_copy(src_hbm.at[ed], src_m.at[slot], msems.at[slot])
        pltpu.async_copy(dst_hbm.at[ed], dst_m.at[slot], msems.at[slot])
        pltpu.async_copy(w_hbm.at[ed], w_m.at[slot], msems.at[slot])

    def m_wait(j, slot):
        ed = pl.ds(wid * EPW + j * CHUNK, CHUNK)
        pltpu.make_async_copy(src_hbm.at[ed], src_m.at[slot],
                              msems.at[slot]).wait()
        pltpu.make_async_copy(dst_hbm.at[ed], dst_m.at[slot],
                              msems.at[slot]).wait()
        pltpu.make_async_copy(w_hbm.at[ed], w_m.at[slot],
                              msems.at[slot]).wait()

    for j0 in range(4):
        m_start(j0, j0)

    def _zrow(r, _):
        for k in range(D // 16):
            scaled_a[r, pl.ds(k * 16, 16)] = jnp.zeros((16,), jnp.float32)
        return ()
    lax.fori_loop(0, CHUNK, _zrow, ())
    for b in range(RPT // CHUNK):
        pltpu.sync_copy(scaled_a,
                        acc_sh.at[pl.ds(sid * RPT + b * CHUNK, CHUNK)])
    plsc.subcore_barrier()

    def g_copy(slot, rbuf, sem):
        return pltpu.make_async_copy(
            h_hbm.at[src_m.at[slot]], rbuf, sem)

    def s_start(obuf, slot, sem):
        pltpu.async_copy(obuf, acc_sh.at[dst_m.at[slot]], sem, add=True)

    def s_wait(obuf, slot, sem):
        pltpu.make_async_copy(obuf, acc_sh.at[dst_m.at[slot]], sem).wait()

    def scale(rbuf, obuf, slot):
        def _group(g, _):
            w16 = w_m[slot, pl.ds(g * 16, 16)]
            for l in range(16):
                ws = jnp.broadcast_to(w16[l], (16,))
                e = g * 16 + l
                for k in range(D // 16):
                    sl = pl.ds(k * 16, 16)
                    obuf[e, sl] = rbuf[e, sl] * ws
            return ()
        lax.fori_loop(0, CHUNK // 16, _group, ())

    m_wait(0, 0)
    g_copy(0, rows_a, gsem_a).start()
    m_wait(1, 1)
    g_copy(1, rows_b, gsem_b).start()

    def _pair(jj, _):
        c = 2 * jj
        g_copy(c % MS, rows_a, gsem_a).wait()

        @pl.when(jj > 0)
        def _():
            s_wait(scaled_a, (c - 2) % MS, ssem_a)

        @pl.when(c + 4 < NCHUNK)
        def _():
            m_start(c + 4, (c + 4) % MS)
        scale(rows_a, scaled_a, c % MS)
        m_wait(c + 2, (c + 2) % MS)
        g_copy((c + 2) % MS, rows_a, gsem_a).start()
        s_start(scaled_a, c % MS, ssem_a)

        g_copy((c + 1) % MS, rows_b, gsem_b).wait()

        @pl.when(jj > 0)
        def _():
            s_wait(scaled_b, (c - 1) % MS, ssem_b)

        @pl.when(c + 5 < NCHUNK)
        def _():
            m_start(c + 5, (c + 5) % MS)
        scale(rows_b, scaled_b, (c + 1) % MS)

        @pl.when(c + 3 < NCHUNK)
        def _():
            m_wait(c + 3, (c + 3) % MS)
            g_copy((c + 3) % MS, rows_b, gsem_b).start()
        s_start(scaled_b, (c + 1) % MS, ssem_b)
        return ()
    lax.fori_loop(0, NCHUNK // 2, _pair, ())

    last = NCHUNK - 1
    g_copy(last % MS, rows_a, gsem_a).wait()
    s_wait(scaled_a, (last - 2) % MS, ssem_a)
    scale(rows_a, scaled_a, last % MS)
    s_start(scaled_a, last % MS, ssem_a)
    s_wait(scaled_b, (last - 1) % MS, ssem_b)
    s_wait(scaled_a, last % MS, ssem_a)

    plsc.subcore_barrier()
    pltpu.sync_copy(acc_sh.at[pl.ds(sid * RPT, RPT)],
                    p_hbm.at[pl.ds(cid * POFF + sid * RPT, RPT)])


def _spmm_sc(h, src, dst, w):
    mesh = plsc.VectorSubcoreMesh(core_axis_name="c", subcore_axis_name="s")
    return pl.kernel(
        _spmm_body,
        out_type=jax.ShapeDtypeStruct((2 * POFF, D), jnp.float32),
        mesh=mesh,
        compiler_params=pltpu.CompilerParams(needs_layout_passes=False),
        scratch_types=[
            pltpu.VMEM((CHUNK, D), jnp.float32),
            pltpu.VMEM((CHUNK, D), jnp.float32),
            pltpu.VMEM((CHUNK, D), jnp.float32),
            pltpu.VMEM((CHUNK, D), jnp.float32),
            pltpu.VMEM((MS, CHUNK), jnp.int32),
            pltpu.VMEM((MS, CHUNK), jnp.int32),
            pltpu.VMEM((MS, CHUNK), jnp.float32),
            pltpu.VMEM_SHARED((P, D), jnp.float32),
            pltpu.SemaphoreType.DMA,
            pltpu.SemaphoreType.DMA,
            pltpu.SemaphoreType.DMA,
            pltpu.SemaphoreType.DMA,
            pltpu.SemaphoreType.DMA((MS,)),
        ],
    )(h, src, dst, w)


def kernel(x, edge_index, edge_weight, W):
    src = edge_index[0]
    dst = edge_index[1]

    h = _matmul(x, W)
    y = _combine(_spmm_sc(h, src, dst, edge_weight))
    out = _combine(_spmm_sc(y, src, dst, edge_weight))
    return out
